# baseline (speedup 1.0000x reference)
import jax
import jax.numpy as jnp
from jax.experimental import pallas as pl
from jax.experimental.pallas import tpu as pltpu

X_SIZE = 128
H_SIZE = 128
TILE = 2000


def _cell_kernel(x_ref, h_ref, c_ref, wx_ref, wh_ref, b_ref, out_ref):
    a = jnp.dot(x_ref[...].astype(jnp.bfloat16), wx_ref[...].astype(jnp.bfloat16),
                preferred_element_type=jnp.float32)
    a = a + jnp.dot(h_ref[...].astype(jnp.bfloat16), wh_ref[...].astype(jnp.bfloat16),
                    preferred_element_type=jnp.float32)
    a = a + b_ref[...]
    i = jax.nn.sigmoid(a[:, 0:H_SIZE])
    o = jax.nn.sigmoid(a[:, H_SIZE:2 * H_SIZE])
    u = jnp.tanh(a[:, 2 * H_SIZE:3 * H_SIZE])
    f = jax.nn.sigmoid(a[:, 3 * H_SIZE:4 * H_SIZE])
    c_sum = c_ref[:, 0:H_SIZE] + c_ref[:, H_SIZE:2 * H_SIZE]
    c = i * u + f * c_sum
    out_ref[:, 0:H_SIZE] = o * jnp.tanh(c)
    out_ref[:, H_SIZE:2 * H_SIZE] = c


def kernel(x, h_child, c_child, W_iou_left, W_iou_right, W_f_left, W_f_right,
           U_iou, b_iou, U_f_w, U_f_b):
    p = x.shape[0]
    d = H_SIZE
    wx = jnp.concatenate([W_iou_left + W_iou_right, W_f_left + W_f_right], axis=1)
    wh = jnp.concatenate([U_iou, U_f_w[:, :d] + U_f_w[:, d:]], axis=1)
    bias = jnp.concatenate([b_iou, (U_f_b[:d] + U_f_b[d:])[None, :]], axis=1)
    h2 = h_child.reshape(p, 2 * d)
    c2 = c_child.reshape(p, 2 * d)

    grid = (p // TILE,)
    return pl.pallas_call(
        _cell_kernel,
        grid=grid,
        in_specs=[
            pl.BlockSpec((TILE, X_SIZE), lambda i: (i, 0)),
            pl.BlockSpec((TILE, 2 * d), lambda i: (i, 0)),
            pl.BlockSpec((TILE, 2 * d), lambda i: (i, 0)),
            pl.BlockSpec((X_SIZE, 4 * d), lambda i: (0, 0)),
            pl.BlockSpec((2 * d, 4 * d), lambda i: (0, 0)),
            pl.BlockSpec((1, 4 * d), lambda i: (0, 0)),
        ],
        out_specs=pl.BlockSpec((TILE, 2 * d), lambda i: (i, 0)),
        out_shape=jax.ShapeDtypeStruct((p, 2 * d), jnp.float32),
        compiler_params=pltpu.CompilerParams(
            dimension_semantics=("parallel",),
        ),
    )(x, h2, c2, wx, wh, bias)


# speedup vs baseline: 1.0321x; 1.0321x over previous
import jax
import jax.numpy as jnp
from jax.experimental import pallas as pl
from jax.experimental.pallas import tpu as pltpu

X_SIZE = 128
H_SIZE = 128
TILE = 4000


def _cell_kernel(x_ref, h_ref, c_ref, wx_ref, wh_ref, b_ref, out_ref):
    a = jnp.dot(x_ref[...].astype(jnp.bfloat16), wx_ref[...].astype(jnp.bfloat16),
                preferred_element_type=jnp.float32)
    a = a + jnp.dot(h_ref[...].astype(jnp.bfloat16), wh_ref[...].astype(jnp.bfloat16),
                    preferred_element_type=jnp.float32)
    a = a + b_ref[...]
    i = jax.nn.sigmoid(a[:, 0:H_SIZE])
    o = jax.nn.sigmoid(a[:, H_SIZE:2 * H_SIZE])
    u = jnp.tanh(a[:, 2 * H_SIZE:3 * H_SIZE])
    f = jax.nn.sigmoid(a[:, 3 * H_SIZE:4 * H_SIZE])
    c_sum = c_ref[:, 0:H_SIZE] + c_ref[:, H_SIZE:2 * H_SIZE]
    c = i * u + f * c_sum
    out_ref[:, 0:H_SIZE] = o * jnp.tanh(c)
    out_ref[:, H_SIZE:2 * H_SIZE] = c


def kernel(x, h_child, c_child, W_iou_left, W_iou_right, W_f_left, W_f_right,
           U_iou, b_iou, U_f_w, U_f_b):
    p = x.shape[0]
    d = H_SIZE
    wx = jnp.concatenate([W_iou_left + W_iou_right, W_f_left + W_f_right], axis=1)
    wh = jnp.concatenate([U_iou, U_f_w[:, :d] + U_f_w[:, d:]], axis=1)
    bias = jnp.concatenate([b_iou, (U_f_b[:d] + U_f_b[d:])[None, :]], axis=1)
    h2 = h_child.reshape(p, 2 * d)
    c2 = c_child.reshape(p, 2 * d)

    grid = (p // TILE,)
    return pl.pallas_call(
        _cell_kernel,
        grid=grid,
        in_specs=[
            pl.BlockSpec((TILE, X_SIZE), lambda i: (i, 0)),
            pl.BlockSpec((TILE, 2 * d), lambda i: (i, 0)),
            pl.BlockSpec((TILE, 2 * d), lambda i: (i, 0)),
            pl.BlockSpec((X_SIZE, 4 * d), lambda i: (0, 0)),
            pl.BlockSpec((2 * d, 4 * d), lambda i: (0, 0)),
            pl.BlockSpec((1, 4 * d), lambda i: (0, 0)),
        ],
        out_specs=pl.BlockSpec((TILE, 2 * d), lambda i: (i, 0)),
        out_shape=jax.ShapeDtypeStruct((p, 2 * d), jnp.float32),
        compiler_params=pltpu.CompilerParams(
            dimension_semantics=("parallel",),
        ),
    )(x, h2, c2, wx, wh, bias)


# speedup vs baseline: 1.0365x; 1.0043x over previous
import jax
import jax.numpy as jnp
from jax.experimental import pallas as pl
from jax.experimental.pallas import tpu as pltpu

X_SIZE = 128
H_SIZE = 128
TILE = 5000


def _cell_kernel(x_ref, h_ref, c_ref, wx_ref, wh_ref, b_ref, out_ref):
    a = jnp.dot(x_ref[...].astype(jnp.bfloat16), wx_ref[...].astype(jnp.bfloat16),
                preferred_element_type=jnp.float32)
    a = a + jnp.dot(h_ref[...].astype(jnp.bfloat16), wh_ref[...].astype(jnp.bfloat16),
                    preferred_element_type=jnp.float32)
    a = a + b_ref[...]
    i = jax.nn.sigmoid(a[:, 0:H_SIZE])
    o = jax.nn.sigmoid(a[:, H_SIZE:2 * H_SIZE])
    u = jnp.tanh(a[:, 2 * H_SIZE:3 * H_SIZE])
    f = jax.nn.sigmoid(a[:, 3 * H_SIZE:4 * H_SIZE])
    c_sum = c_ref[:, 0:H_SIZE] + c_ref[:, H_SIZE:2 * H_SIZE]
    c = i * u + f * c_sum
    out_ref[:, 0:H_SIZE] = o * jnp.tanh(c)
    out_ref[:, H_SIZE:2 * H_SIZE] = c


def kernel(x, h_child, c_child, W_iou_left, W_iou_right, W_f_left, W_f_right,
           U_iou, b_iou, U_f_w, U_f_b):
    p = x.shape[0]
    d = H_SIZE
    wx = jnp.concatenate([W_iou_left + W_iou_right, W_f_left + W_f_right], axis=1)
    wh = jnp.concatenate([U_iou, U_f_w[:, :d] + U_f_w[:, d:]], axis=1)
    bias = jnp.concatenate([b_iou, (U_f_b[:d] + U_f_b[d:])[None, :]], axis=1)
    h2 = h_child.reshape(p, 2 * d)
    c2 = c_child.reshape(p, 2 * d)

    grid = (p // TILE,)
    return pl.pallas_call(
        _cell_kernel,
        grid=grid,
        in_specs=[
            pl.BlockSpec((TILE, X_SIZE), lambda i: (i, 0)),
            pl.BlockSpec((TILE, 2 * d), lambda i: (i, 0)),
            pl.BlockSpec((TILE, 2 * d), lambda i: (i, 0)),
            pl.BlockSpec((X_SIZE, 4 * d), lambda i: (0, 0)),
            pl.BlockSpec((2 * d, 4 * d), lambda i: (0, 0)),
            pl.BlockSpec((1, 4 * d), lambda i: (0, 0)),
        ],
        out_specs=pl.BlockSpec((TILE, 2 * d), lambda i: (i, 0)),
        out_shape=jax.ShapeDtypeStruct((p, 2 * d), jnp.float32),
        compiler_params=pltpu.CompilerParams(
            dimension_semantics=("parallel",),
        ),
    )(x, h2, c2, wx, wh, bias)


# speedup vs baseline: 1.0404x; 1.0037x over previous
import jax
import jax.numpy as jnp
from jax.experimental import pallas as pl
from jax.experimental.pallas import tpu as pltpu

X_SIZE = 128
H_SIZE = 128
TILE = 5000


def _cell_kernel(x_ref, h_ref, c_ref, wx_ref, wh_ref, b_ref, out_ref):
    a = jnp.dot(x_ref[...].astype(jnp.bfloat16), wx_ref[...].astype(jnp.bfloat16),
                preferred_element_type=jnp.float32)
    a = a + jnp.dot(h_ref[...].astype(jnp.bfloat16), wh_ref[...].astype(jnp.bfloat16),
                    preferred_element_type=jnp.float32)
    a = a + b_ref[...]
    def _sig(v):
        return 0.5 * jnp.tanh(0.5 * v) + 0.5

    i = _sig(a[:, 0:H_SIZE])
    o = _sig(a[:, H_SIZE:2 * H_SIZE])
    u = jnp.tanh(a[:, 2 * H_SIZE:3 * H_SIZE])
    f = _sig(a[:, 3 * H_SIZE:4 * H_SIZE])
    c_sum = c_ref[:, 0:H_SIZE] + c_ref[:, H_SIZE:2 * H_SIZE]
    c = i * u + f * c_sum
    out_ref[:, 0:H_SIZE] = o * jnp.tanh(c)
    out_ref[:, H_SIZE:2 * H_SIZE] = c


def kernel(x, h_child, c_child, W_iou_left, W_iou_right, W_f_left, W_f_right,
           U_iou, b_iou, U_f_w, U_f_b):
    p = x.shape[0]
    d = H_SIZE
    wx = jnp.concatenate([W_iou_left + W_iou_right, W_f_left + W_f_right], axis=1)
    wh = jnp.concatenate([U_iou, U_f_w[:, :d] + U_f_w[:, d:]], axis=1)
    bias = jnp.concatenate([b_iou, (U_f_b[:d] + U_f_b[d:])[None, :]], axis=1)
    h2 = h_child.reshape(p, 2 * d)
    c2 = c_child.reshape(p, 2 * d)

    grid = (p // TILE,)
    return pl.pallas_call(
        _cell_kernel,
        grid=grid,
        in_specs=[
            pl.BlockSpec((TILE, X_SIZE), lambda i: (i, 0)),
            pl.BlockSpec((TILE, 2 * d), lambda i: (i, 0)),
            pl.BlockSpec((TILE, 2 * d), lambda i: (i, 0)),
            pl.BlockSpec((X_SIZE, 4 * d), lambda i: (0, 0)),
            pl.BlockSpec((2 * d, 4 * d), lambda i: (0, 0)),
            pl.BlockSpec((1, 4 * d), lambda i: (0, 0)),
        ],
        out_specs=pl.BlockSpec((TILE, 2 * d), lambda i: (i, 0)),
        out_shape=jax.ShapeDtypeStruct((p, 2 * d), jnp.float32),
        compiler_params=pltpu.CompilerParams(
            dimension_semantics=("parallel",),
        ),
    )(x, h2, c2, wx, wh, bias)
